# baseline (speedup 1.0000x reference)
import functools

import jax
import jax.numpy as jnp
from jax import lax
from jax.experimental import pallas as pl
from jax.experimental.pallas import tpu as pltpu, tpu_sc as plsc

N = 10000
E = 320000
D = 128
L = 3

NC = 2
NS = 16
NW = NC * NS
EPW = E // NW
CH = 80
NCHUNK = EPW // CH
ZR = 40
NZCH = N // ZR


def _ee_body(a_ref, w_ref, b_ref, o_ref):
    o_ref[...] = (
        jnp.dot(a_ref[...], w_ref[...], preferred_element_type=jnp.float32)
        + b_ref[...]
    )


def _edge_embed(attr8, w8, b):
    BE = 8000
    return pl.pallas_call(
        _ee_body,
        grid=(E // BE,),
        in_specs=[
            pl.BlockSpec((BE, 8), lambda i: (i, 0)),
            pl.BlockSpec((8, D), lambda i: (0, 0)),
            pl.BlockSpec((1, D), lambda i: (0, 0)),
        ],
        out_specs=pl.BlockSpec((BE, D), lambda i: (i, 0)),
        out_shape=jax.ShapeDtypeStruct((E, D), jnp.float32),
    )(attr8, w8, b)


def _sc_agg_body(h_hbm, ee_hbm, row_hbm, col_hbm, out_hbm,
                 ridx_v, cidx_v, rows_v, ee_v, zb_v, agg_sh, gsem, esem):
    c = lax.axis_index("c")
    s = lax.axis_index("s")
    wid = s * NC + c
    ebase = wid * EPW

    def _zero_body(i, _):
        for j in range(8):
            zb_v[i, pl.ds(j * 16, 16)] = jnp.zeros((16,), jnp.float32)
        return 0

    lax.fori_loop(0, ZR, _zero_body, 0)
    for k in range((NZCH + NS - 1) // NS):
        idx = s + k * NS

        @pl.when(idx < NZCH)
        def _():
            pltpu.sync_copy(zb_v, agg_sh.at[pl.ds(idx * ZR, ZR)])

    plsc.subcore_barrier()

    def _idx_copy(t, slot):
        base = ebase + t * CH
        pltpu.sync_copy(row_hbm.at[pl.ds(base, CH)], ridx_v.at[slot])
        pltpu.sync_copy(col_hbm.at[pl.ds(base, CH)], cidx_v.at[slot])

    def _issue(t, slot):
        base = ebase + t * CH
        pltpu.async_copy(h_hbm.at[ridx_v.at[slot]], rows_v.at[slot],
                         gsem.at[slot])
        pltpu.async_copy(ee_hbm.at[pl.ds(base, CH)], ee_v.at[slot],
                         esem.at[slot])

    _idx_copy(0, 0)
    _issue(0, 0)
    _idx_copy(1, 1)

    def _chunk_body(t, _):
        p = lax.rem(t, 2)
        q = 1 - p

        @pl.when(t + 1 < NCHUNK)
        def _():
            _issue(t + 1, q)

        base = ebase + t * CH
        pltpu.make_async_copy(h_hbm.at[ridx_v.at[p]], rows_v.at[p],
                              gsem.at[p]).wait()
        pltpu.make_async_copy(ee_hbm.at[pl.ds(base, CH)], ee_v.at[p],
                              esem.at[p]).wait()

        def _edge_body(e, _):
            for j in range(8):
                hv = rows_v[p, e, pl.ds(j * 16, 16)]
                ev = ee_v[p, e, pl.ds(j * 16, 16)]
                rows_v[p, e, pl.ds(j * 16, 16)] = jnp.maximum(hv + ev, 0.0)
            return 0

        lax.fori_loop(0, CH, _edge_body, 0)
        pltpu.sync_copy(rows_v.at[p], agg_sh.at[cidx_v.at[p]], add=True)

        @pl.when(t + 2 < NCHUNK)
        def _():
            _idx_copy(t + 2, p)

        return 0

    lax.fori_loop(0, NCHUNK, _chunk_body, 0)
    plsc.subcore_barrier()

    for k in range((NZCH + NS - 1) // NS):
        idx = s + k * NS

        @pl.when(idx < NZCH)
        def _():
            r0 = idx * ZR
            pltpu.sync_copy(agg_sh.at[pl.ds(r0, ZR)], zb_v)
            pltpu.sync_copy(zb_v, out_hbm.at[c].at[pl.ds(r0, ZR)])


@functools.lru_cache(maxsize=None)
def _make_sc_agg():
    return pl.kernel(
        _sc_agg_body,
        out_type=jax.ShapeDtypeStruct((NC, N, D), jnp.float32),
        mesh=plsc.VectorSubcoreMesh(
            core_axis_name="c", subcore_axis_name="s",
            num_cores=NC, num_subcores=NS,
        ),
        scratch_types=[
            pltpu.VMEM((2, CH), jnp.int32),
            pltpu.VMEM((2, CH), jnp.int32),
            pltpu.VMEM((2, CH, D), jnp.float32),
            pltpu.VMEM((2, CH, D), jnp.float32),
            pltpu.VMEM((ZR, D), jnp.float32),
            pltpu.VMEM_SHARED((N, D), jnp.float32),
            pltpu.SemaphoreType.DMA((2,)),
            pltpu.SemaphoreType.DMA((2,)),
        ],
    )


def _mlp_body(eps_ref, h_ref, agg_ref, w1_ref, b1_ref, gm_ref, bm_ref,
              w2_ref, b2_ref, g_ref, be_ref, o_ref, *, final):
    z = (1.0 + eps_ref[0]) * h_ref[...] + agg_ref[0] + agg_ref[1]
    y = jnp.dot(z, w1_ref[...], preferred_element_type=jnp.float32) + b1_ref[...]
    mu = jnp.mean(y, axis=0, keepdims=True)
    var = jnp.mean((y - mu) ** 2, axis=0, keepdims=True)
    y = (y - mu) / jnp.sqrt(var + 1e-5) * gm_ref[...] + bm_ref[...]
    y = jnp.maximum(y, 0.0)
    y2 = jnp.dot(y, w2_ref[...], preferred_element_type=jnp.float32) + b2_ref[...]
    mu2 = jnp.mean(y2, axis=0, keepdims=True)
    var2 = jnp.mean((y2 - mu2) ** 2, axis=0, keepdims=True)
    y2 = (y2 - mu2) / jnp.sqrt(var2 + 1e-5) * g_ref[...] + be_ref[...]
    if not final:
        y2 = jnp.maximum(y2, 0.0)
    o_ref[...] = y2


def _mlp(eps, h, agg, w1, b1, gm, bm, w2, b2, g, be, final):
    body = functools.partial(_mlp_body, final=final)
    return pl.pallas_call(
        body,
        in_specs=[
            pl.BlockSpec(memory_space=pltpu.SMEM),
            pl.BlockSpec((N, D), lambda: (0, 0)),
            pl.BlockSpec((NC, N, D), lambda: (0, 0, 0)),
            pl.BlockSpec((D, 2 * D), lambda: (0, 0)),
            pl.BlockSpec((1, 2 * D), lambda: (0, 0)),
            pl.BlockSpec((1, 2 * D), lambda: (0, 0)),
            pl.BlockSpec((1, 2 * D), lambda: (0, 0)),
            pl.BlockSpec((2 * D, D), lambda: (0, 0)),
            pl.BlockSpec((1, D), lambda: (0, 0)),
            pl.BlockSpec((1, D), lambda: (0, 0)),
            pl.BlockSpec((1, D), lambda: (0, 0)),
        ],
        out_specs=pl.BlockSpec((N, D), lambda: (0, 0)),
        out_shape=jax.ShapeDtypeStruct((N, D), jnp.float32),
    )(eps, h, agg, w1, b1, gm, bm, w2, b2, g, be)


def kernel(x, edge_index, edge_attr, batch, node_enc_W, edge_W, edge_b,
           W1, b1, g_mid, be_mid, W2, b2, eps_arr, gamma, beta):
    h = jnp.take(node_enc_W, x, axis=0)
    row = edge_index[0]
    col = edge_index[1]
    attr8 = jnp.pad(edge_attr, ((0, 0), (0, 1)))

    for l in range(L):
        w8 = jnp.pad(edge_W[l], ((0, 1), (0, 0)))
        ee = _edge_embed(attr8, w8, edge_b[l].reshape(1, D))
        agg = _make_sc_agg()(h, ee, row, col)
        h = _mlp(
            eps_arr[l].reshape(1), h, agg,
            W1[l], b1[l].reshape(1, 2 * D),
            g_mid[l].reshape(1, 2 * D), be_mid[l].reshape(1, 2 * D),
            W2[l], b2[l].reshape(1, D),
            gamma[l].reshape(1, D), beta[l].reshape(1, D),
            final=(l == L - 1),
        )
    return h


# speedup vs baseline: 1.9185x; 1.9185x over previous
import functools

import jax
import jax.numpy as jnp
from jax import lax
from jax.experimental import pallas as pl
from jax.experimental.pallas import tpu as pltpu, tpu_sc as plsc

N = 10000
E = 320000
D = 128
L = 3

NC = 2
NS = 16
NW = NC * NS
EPW = E // NW
CH = 80
NCHUNK = EPW // CH
ZR = 40
NZCH = N // ZR


def _ee_body(a_ref, w_ref, b_ref, o_ref):
    o_ref[...] = (
        jnp.dot(a_ref[...], w_ref[...], preferred_element_type=jnp.float32)
        + b_ref[...]
    )


def _edge_embed(attr8, w8, b):
    BE = 8000
    return pl.pallas_call(
        _ee_body,
        grid=(E // BE,),
        in_specs=[
            pl.BlockSpec((BE, 8), lambda i: (i, 0)),
            pl.BlockSpec((8, D), lambda i: (0, 0)),
            pl.BlockSpec((1, D), lambda i: (0, 0)),
        ],
        out_specs=pl.BlockSpec((BE, D), lambda i: (i, 0)),
        out_shape=jax.ShapeDtypeStruct((E, D), jnp.float32),
    )(attr8, w8, b)


def _sc_agg_body(h_hbm, ee_hbm, row_hbm, col_hbm, out_hbm,
                 ridx0_v, ridx1_v, cidx0_v, cidx1_v, rows0_v, rows1_v,
                 ee0_v, ee1_v, zb_v, agg_sh,
                 gsem0, gsem1, esem0, esem1):
    c = lax.axis_index("c")
    s = lax.axis_index("s")
    wid = s * NC + c
    ebase = wid * EPW
    ridx = (ridx0_v, ridx1_v)
    cidx = (cidx0_v, cidx1_v)
    rows = (rows0_v, rows1_v)
    ees = (ee0_v, ee1_v)
    gsem = (gsem0, gsem1)
    esem = (esem0, esem1)

    def _zero_body(i, _):
        for j in range(8):
            zb_v[i, pl.ds(j * 16, 16)] = jnp.zeros((16,), jnp.float32)
        return 0

    lax.fori_loop(0, ZR, _zero_body, 0)
    for k in range((NZCH + NS - 1) // NS):
        idx = s + k * NS

        @pl.when(idx < NZCH)
        def _():
            pltpu.sync_copy(zb_v, agg_sh.at[pl.ds(idx * ZR, ZR)])

    plsc.subcore_barrier()

    def _idx_copy(t, slot):
        base = ebase + t * CH
        pltpu.sync_copy(row_hbm.at[pl.ds(base, CH)], ridx[slot])
        pltpu.sync_copy(col_hbm.at[pl.ds(base, CH)], cidx[slot])

    def _issue(t, slot):
        base = ebase + t * CH
        pltpu.async_copy(h_hbm.at[ridx[slot]], rows[slot], gsem[slot])
        pltpu.async_copy(ee_hbm.at[pl.ds(base, CH)], ees[slot], esem[slot])

    def _wait(slot):
        pltpu.make_async_copy(h_hbm.at[ridx[slot]], rows[slot],
                              gsem[slot]).wait()
        pltpu.make_async_copy(ee_hbm.at[pl.ds(0, CH)], ees[slot],
                              esem[slot]).wait()

    def _compute_scatter(slot):
        rv = rows[slot]
        ev = ees[slot]

        def _edge_body(e, _):
            for j in range(8):
                sl = pl.ds(j * 16, 16)
                rv[e, sl] = jnp.maximum(rv[e, sl] + ev[e, sl], 0.0)
            return 0

        lax.fori_loop(0, CH, _edge_body, 0)
        pltpu.sync_copy(rv, agg_sh.at[cidx[slot]], add=True)

    _idx_copy(0, 0)
    _issue(0, 0)
    _idx_copy(1, 1)

    NPAIR = (NCHUNK - 1) // 2

    def _pair_body(i, _):
        t = 2 * i
        _issue(t + 1, 1)
        _wait(0)
        _compute_scatter(0)
        _idx_copy(t + 2, 0)
        _issue(t + 2, 0)
        _wait(1)
        _compute_scatter(1)

        @pl.when(t + 3 < NCHUNK)
        def _():
            _idx_copy(t + 3, 1)

        return 0

    lax.fori_loop(0, NPAIR, _pair_body, 0)
    _wait(0)
    _compute_scatter(0)
    plsc.subcore_barrier()

    for k in range((NZCH + NS - 1) // NS):
        idx = s + k * NS

        @pl.when(idx < NZCH)
        def _():
            r0 = idx * ZR
            pltpu.sync_copy(agg_sh.at[pl.ds(r0, ZR)], zb_v)
            pltpu.sync_copy(zb_v, out_hbm.at[c].at[pl.ds(r0, ZR)])


@functools.lru_cache(maxsize=None)
def _make_sc_agg():
    return pl.kernel(
        _sc_agg_body,
        out_type=jax.ShapeDtypeStruct((NC, N, D), jnp.float32),
        mesh=plsc.VectorSubcoreMesh(
            core_axis_name="c", subcore_axis_name="s",
            num_cores=NC, num_subcores=NS,
        ),
        scratch_types=[
            pltpu.VMEM((CH,), jnp.int32),
            pltpu.VMEM((CH,), jnp.int32),
            pltpu.VMEM((CH,), jnp.int32),
            pltpu.VMEM((CH,), jnp.int32),
            pltpu.VMEM((CH, D), jnp.float32),
            pltpu.VMEM((CH, D), jnp.float32),
            pltpu.VMEM((CH, D), jnp.float32),
            pltpu.VMEM((CH, D), jnp.float32),
            pltpu.VMEM((ZR, D), jnp.float32),
            pltpu.VMEM_SHARED((N, D), jnp.float32),
            pltpu.SemaphoreType.DMA,
            pltpu.SemaphoreType.DMA,
            pltpu.SemaphoreType.DMA,
            pltpu.SemaphoreType.DMA,
        ],
    )


def _mlp_body(eps_ref, h_ref, agg_ref, w1_ref, b1_ref, gm_ref, bm_ref,
              w2_ref, b2_ref, g_ref, be_ref, o_ref, *, final):
    z = (1.0 + eps_ref[0]) * h_ref[...] + agg_ref[0] + agg_ref[1]
    y = jnp.dot(z, w1_ref[...], preferred_element_type=jnp.float32) + b1_ref[...]
    mu = jnp.mean(y, axis=0, keepdims=True)
    var = jnp.mean((y - mu) ** 2, axis=0, keepdims=True)
    y = (y - mu) / jnp.sqrt(var + 1e-5) * gm_ref[...] + bm_ref[...]
    y = jnp.maximum(y, 0.0)
    y2 = jnp.dot(y, w2_ref[...], preferred_element_type=jnp.float32) + b2_ref[...]
    mu2 = jnp.mean(y2, axis=0, keepdims=True)
    var2 = jnp.mean((y2 - mu2) ** 2, axis=0, keepdims=True)
    y2 = (y2 - mu2) / jnp.sqrt(var2 + 1e-5) * g_ref[...] + be_ref[...]
    if not final:
        y2 = jnp.maximum(y2, 0.0)
    o_ref[...] = y2


def _mlp(eps, h, agg, w1, b1, gm, bm, w2, b2, g, be, final):
    body = functools.partial(_mlp_body, final=final)
    return pl.pallas_call(
        body,
        in_specs=[
            pl.BlockSpec(memory_space=pltpu.SMEM),
            pl.BlockSpec((N, D), lambda: (0, 0)),
            pl.BlockSpec((NC, N, D), lambda: (0, 0, 0)),
            pl.BlockSpec((D, 2 * D), lambda: (0, 0)),
            pl.BlockSpec((1, 2 * D), lambda: (0, 0)),
            pl.BlockSpec((1, 2 * D), lambda: (0, 0)),
            pl.BlockSpec((1, 2 * D), lambda: (0, 0)),
            pl.BlockSpec((2 * D, D), lambda: (0, 0)),
            pl.BlockSpec((1, D), lambda: (0, 0)),
            pl.BlockSpec((1, D), lambda: (0, 0)),
            pl.BlockSpec((1, D), lambda: (0, 0)),
        ],
        out_specs=pl.BlockSpec((N, D), lambda: (0, 0)),
        out_shape=jax.ShapeDtypeStruct((N, D), jnp.float32),
    )(eps, h, agg, w1, b1, gm, bm, w2, b2, g, be)


def kernel(x, edge_index, edge_attr, batch, node_enc_W, edge_W, edge_b,
           W1, b1, g_mid, be_mid, W2, b2, eps_arr, gamma, beta):
    h = jnp.take(node_enc_W, x, axis=0)
    row = edge_index[0]
    col = edge_index[1]
    attr8 = jnp.pad(edge_attr, ((0, 0), (0, 1)))

    for l in range(L):
        w8 = jnp.pad(edge_W[l], ((0, 1), (0, 0)))
        ee = _edge_embed(attr8, w8, edge_b[l].reshape(1, D))
        agg = _make_sc_agg()(h, ee, row, col)
        h = _mlp(
            eps_arr[l].reshape(1), h, agg,
            W1[l], b1[l].reshape(1, 2 * D),
            g_mid[l].reshape(1, 2 * D), be_mid[l].reshape(1, 2 * D),
            W2[l], b2[l].reshape(1, D),
            gamma[l].reshape(1, D), beta[l].reshape(1, D),
            final=(l == L - 1),
        )
    return h
